# Initial kernel scaffold; baseline (speedup 1.0000x reference)
import jax
import jax.numpy as jnp
from jax.experimental import pallas as pl


def kernel(pop_prob, items):
    raise NotImplementedError("write your pallas kernel here")


# speedup vs baseline: 133.9727x; 133.9727x over previous
import functools

import jax
import jax.numpy as jnp
from jax import lax
from jax.experimental import pallas as pl
from jax.experimental.pallas import tpu as pltpu
from jax.experimental.pallas import tpu_sc as plsc

_VOCAB1 = 1000001
_VOCAB_PAD = 1000448
_ROWS = _VOCAB_PAD // 128
_ROW_BLK = _ROWS // 8

_N = 16384 * 200
_NW = 32
_PER_W = _N // _NW
_CHUNK = 12800
_NCH = _PER_W // _CHUNK


def _log_body(p_ref, o_ref):
    o_ref[...] = jnp.log(p_ref[...])


def _log_table(table_2d):
    return pl.pallas_call(
        _log_body,
        out_shape=jax.ShapeDtypeStruct((_ROWS, 128), jnp.float32),
    )(table_2d)


_MESH = plsc.VectorSubcoreMesh(core_axis_name="c", subcore_axis_name="s")


@functools.partial(
    pl.kernel,
    out_type=jax.ShapeDtypeStruct((_N,), jnp.float32),
    mesh=_MESH,
    scratch_types=[
        pltpu.VMEM((_CHUNK,), jnp.int32),
        pltpu.VMEM((_CHUNK,), jnp.float32),
        pltpu.SemaphoreType.DMA,
    ],
)
def _sc_gather(table_hbm, idx_hbm, out_hbm, idx_v, val_v, sem):
    wid = lax.axis_index("s") * 2 + lax.axis_index("c")
    base = wid * _PER_W

    def body(i, _):
        off = base + i * _CHUNK
        pltpu.sync_copy(idx_hbm.at[pl.ds(off, _CHUNK)], idx_v)
        pltpu.async_copy(table_hbm.at[idx_v], val_v, sem).wait()
        pltpu.sync_copy(val_v, out_hbm.at[pl.ds(off, _CHUNK)])
        return 0

    lax.fori_loop(0, _NCH, body, 0)


def kernel(pop_prob, items):
    table = jnp.pad(pop_prob, (0, _VOCAB_PAD - _VOCAB1), constant_values=1.0)
    logt = _log_table(table.reshape(_ROWS, 128)).reshape(-1)
    idx = items.reshape(-1).astype(jnp.int32)
    out = _sc_gather(logt, idx)
    return out.reshape(items.shape)


# speedup vs baseline: 134.0032x; 1.0002x over previous
import functools

import jax
import jax.numpy as jnp
from jax import lax
from jax.experimental import pallas as pl
from jax.experimental.pallas import tpu as pltpu
from jax.experimental.pallas import tpu_sc as plsc

_VOCAB1 = 1000001
_VOCAB_PAD = 1000448
_ROWS = _VOCAB_PAD // 128
_ROW_BLK = _ROWS // 8

_B = 16384
_H = 200
_NW = 32
_ROWS_W = _B // _NW
_CROWS = 64
_NCH = _ROWS_W // _CROWS


def _log_body(p_ref, o_ref):
    o_ref[...] = jnp.log(p_ref[...])


def _log_table(table_2d):
    return pl.pallas_call(
        _log_body,
        out_shape=jax.ShapeDtypeStruct((_ROWS, 128), jnp.float32),
    )(table_2d)


_MESH = plsc.VectorSubcoreMesh(core_axis_name="c", subcore_axis_name="s")


@functools.partial(
    pl.kernel,
    out_type=jax.ShapeDtypeStruct((_B, _H), jnp.float32),
    mesh=_MESH,
    scratch_types=[
        pltpu.VMEM((_CROWS, _H), jnp.int32),
        pltpu.VMEM((_CROWS, _H), jnp.float32),
        pltpu.SemaphoreType.DMA,
    ],
    compiler_params=pltpu.CompilerParams(use_tc_tiling_on_sc=False),
)
def _sc_gather(table_hbm, idx_hbm, out_hbm, idx_v, val_v, sem):
    wid = lax.axis_index("s") * 2 + lax.axis_index("c")
    base = wid * _ROWS_W

    def body(i, _):
        r0 = base + i * _CROWS
        pltpu.sync_copy(idx_hbm.at[pl.ds(r0, _CROWS)], idx_v)

        def row(k, _):
            pltpu.async_copy(table_hbm.at[idx_v.at[k]], val_v.at[k], sem)
            return 0

        lax.fori_loop(0, _CROWS, row, 0)
        pltpu.make_async_copy(out_hbm.at[pl.ds(r0, _CROWS)], val_v, sem).wait()
        pltpu.sync_copy(val_v, out_hbm.at[pl.ds(r0, _CROWS)])
        return 0

    lax.fori_loop(0, _NCH, body, 0)


def kernel(pop_prob, items):
    table = jnp.pad(pop_prob, (0, _VOCAB_PAD - _VOCAB1), constant_values=1.0)
    logt = _log_table(table.reshape(_ROWS, 128)).reshape(-1)
    idx = items.astype(jnp.int32)
    return _sc_gather(logt, idx)


# speedup vs baseline: 210.3270x; 1.5696x over previous
import functools

import jax
import jax.numpy as jnp
from jax import lax
from jax.experimental import pallas as pl
from jax.experimental.pallas import tpu as pltpu
from jax.experimental.pallas import tpu_sc as plsc

_VOCAB1 = 1000001
_VOCAB_PAD = 1000448
_ROWS = _VOCAB_PAD // 128

_N = 16384 * 200
_NW = 32
_PER_W = _N // _NW
_CHUNK = 12800
_NCH = _PER_W // _CHUNK


def _log_body(p_ref, o_ref):
    o_ref[...] = jnp.log(p_ref[...])


def _log_table(table_2d):
    return pl.pallas_call(
        _log_body,
        out_shape=jax.ShapeDtypeStruct((_ROWS, 128), jnp.float32),
    )(table_2d)


_MESH = plsc.VectorSubcoreMesh(core_axis_name="c", subcore_axis_name="s")


@functools.partial(
    pl.kernel,
    out_type=jax.ShapeDtypeStruct((_N,), jnp.float32),
    mesh=_MESH,
    scratch_types=[
        pltpu.VMEM((_CHUNK,), jnp.int32),
        pltpu.VMEM((_CHUNK,), jnp.float32),
        pltpu.VMEM_SHARED((_VOCAB_PAD,), jnp.float32),
        pltpu.SemaphoreType.DMA,
    ],
)
def _sc_gather(table_hbm, idx_hbm, out_hbm, idx_v, val_v, table_sp, sem):
    sid = lax.axis_index("s")
    wid = sid * 2 + lax.axis_index("c")
    base = wid * _PER_W

    @pl.when(sid == 0)
    def _():
        pltpu.sync_copy(table_hbm, table_sp)

    plsc.subcore_barrier()

    def body(i, _):
        off = base + i * _CHUNK
        pltpu.sync_copy(idx_hbm.at[pl.ds(off, _CHUNK)], idx_v)
        pltpu.async_copy(table_sp.at[idx_v], val_v, sem).wait()
        pltpu.sync_copy(val_v, out_hbm.at[pl.ds(off, _CHUNK)])
        return 0

    lax.fori_loop(0, _NCH, body, 0)


def kernel(pop_prob, items):
    table = jnp.pad(pop_prob, (0, _VOCAB_PAD - _VOCAB1), constant_values=1.0)
    logt = _log_table(table.reshape(_ROWS, 128)).reshape(-1)
    idx = items.reshape(-1).astype(jnp.int32)
    out = _sc_gather(logt, idx)
    return out.reshape(items.shape)
